# Initial kernel scaffold; baseline (speedup 1.0000x reference)
import jax
import jax.numpy as jnp
from jax.experimental import pallas as pl


def kernel(node_features, mesh_edge_features, senders, receivers, edge_params, node_params):
    raise NotImplementedError("write your pallas kernel here")


# speedup vs baseline: 2.4139x; 2.4139x over previous
import functools

import jax
import jax.numpy as jnp
from jax import lax
from jax.experimental import pallas as pl
from jax.experimental.pallas import tpu as pltpu
from jax.experimental.pallas import tpu_sc as plsc

_NN = 10000
_NE = 320000
_D = 128
_CH = 80
_NC = 2
_NS = 16
_NW = _NC * _NS
_EPW = _NE // _NW
_NCH = _EPW // _CH
_EPC = _NE // _NC
_NNP = 10240
_RPT = _NNP // _NS


def _premul_body(n_ref, w_ref, p_ref, q_ref):
    n = n_ref[...]
    p_ref[...] = jnp.dot(n, w_ref[0:_D, :], preferred_element_type=jnp.float32)
    q_ref[...] = jnp.dot(n, w_ref[_D:2 * _D, :], preferred_element_type=jnp.float32)


def _premul(node, w1ab):
    b = 2000
    return pl.pallas_call(
        _premul_body,
        grid=(_NN // b,),
        in_specs=[pl.BlockSpec((b, _D), lambda i: (i, 0)),
                  pl.BlockSpec((2 * _D, _D), lambda i: (0, 0))],
        out_specs=[pl.BlockSpec((b, _D), lambda i: (i, 0)),
                   pl.BlockSpec((b, _D), lambda i: (i, 0))],
        out_shape=[jax.ShapeDtypeStruct((_NN, _D), jnp.float32),
                   jax.ShapeDtypeStruct((_NN, _D), jnp.float32)],
    )(node, w1ab)


def _edge_body(s_ref, r_ref, e_ref, w1c_ref, b1_ref, w2_ref, b2_ref,
               g_ref, bb_ref, u_ref, ne_ref):
    e = e_ref[...]
    x = (s_ref[...] + r_ref[...] + b1_ref[...]
         + jnp.dot(e, w1c_ref[...], preferred_element_type=jnp.float32))
    h = jnp.maximum(x, 0.0)
    o = jnp.dot(h, w2_ref[...], preferred_element_type=jnp.float32) + b2_ref[...]
    mu = jnp.mean(o, axis=-1, keepdims=True)
    oc = o - mu
    var = jnp.mean(oc * oc, axis=-1, keepdims=True)
    u = oc * lax.rsqrt(var + 1e-5) * g_ref[...] + bb_ref[...]
    u_ref[...] = u
    ne_ref[...] = u + e


def _edge_mlp(s_feat, r_feat, e_feat, w1c, b1, w2, b2, ln_g, ln_b):
    b = 512
    row = lambda i: (i, 0)
    rep = lambda i: (0, 0)
    return pl.pallas_call(
        _edge_body,
        grid=(_NE // b,),
        in_specs=[pl.BlockSpec((b, _D), row),
                  pl.BlockSpec((b, _D), row),
                  pl.BlockSpec((b, _D), row),
                  pl.BlockSpec((_D, _D), rep),
                  pl.BlockSpec((1, _D), rep),
                  pl.BlockSpec((_D, _D), rep),
                  pl.BlockSpec((1, _D), rep),
                  pl.BlockSpec((1, _D), rep),
                  pl.BlockSpec((1, _D), rep)],
        out_specs=[pl.BlockSpec((b, _D), row),
                   pl.BlockSpec((b, _D), row)],
        out_shape=[jax.ShapeDtypeStruct((_NE, _D), jnp.float32),
                   jax.ShapeDtypeStruct((_NE, _D), jnp.float32)],
    )(s_feat, r_feat, e_feat, w1c, b1, w2, b2, ln_g, ln_b)


def _node_body(n_ref, a_ref, w1_ref, b1_ref, w2_ref, b2_ref, g_ref, bb_ref,
               o_ref):
    n = n_ref[...]
    a = a_ref[0, :, :] + a_ref[1, :, :]
    x = (jnp.dot(n, w1_ref[0:_D, :], preferred_element_type=jnp.float32)
         + jnp.dot(a, w1_ref[_D:2 * _D, :], preferred_element_type=jnp.float32)
         + b1_ref[...])
    h = jnp.maximum(x, 0.0)
    o = jnp.dot(h, w2_ref[...], preferred_element_type=jnp.float32) + b2_ref[...]
    mu = jnp.mean(o, axis=-1, keepdims=True)
    oc = o - mu
    var = jnp.mean(oc * oc, axis=-1, keepdims=True)
    o_ref[...] = oc * lax.rsqrt(var + 1e-5) * g_ref[...] + bb_ref[...] + n


def _node_mlp(node, agg2, w1, b1, w2, b2, ln_g, ln_b):
    b = 2000
    rep = lambda i: (0, 0)
    return pl.pallas_call(
        _node_body,
        grid=(_NN // b,),
        in_specs=[pl.BlockSpec((b, _D), lambda i: (i, 0)),
                  pl.BlockSpec((2, b, _D), lambda i: (0, i, 0)),
                  pl.BlockSpec((2 * _D, _D), rep),
                  pl.BlockSpec((1, _D), rep),
                  pl.BlockSpec((_D, _D), rep),
                  pl.BlockSpec((1, _D), rep),
                  pl.BlockSpec((1, _D), rep),
                  pl.BlockSpec((1, _D), rep)],
        out_specs=pl.BlockSpec((b, _D), lambda i: (i, 0)),
        out_shape=jax.ShapeDtypeStruct((_NN, _D), jnp.float32),
    )(node, agg2, w1, b1, w2, b2, ln_g, ln_b)


def _sc_gather(p, q, s_idx, r_idx):
    mesh = plsc.VectorSubcoreMesh(core_axis_name="c", subcore_axis_name="s")

    @functools.partial(
        pl.kernel, mesh=mesh,
        out_type=[jax.ShapeDtypeStruct((_NE, _D), jnp.float32),
                  jax.ShapeDtypeStruct((_NE, _D), jnp.float32)],
        scratch_types=[pltpu.VMEM((_CH,), jnp.int32),
                       pltpu.VMEM((_CH,), jnp.int32),
                       pltpu.VMEM((_CH, _D), jnp.float32),
                       pltpu.VMEM((_CH, _D), jnp.float32),
                       pltpu.SemaphoreType.DMA,
                       pltpu.SemaphoreType.DMA],
    )
    def k(p_hbm, q_hbm, s_hbm, r_hbm, so_hbm, ro_hbm,
          si_v, ri_v, ba, bb, sa, sb):
        wid = lax.axis_index("s") * _NC + lax.axis_index("c")
        base = wid * _EPW

        def body(i, carry):
            off = base + i * _CH
            pltpu.sync_copy(s_hbm.at[pl.ds(off, _CH)], si_v)
            pltpu.sync_copy(r_hbm.at[pl.ds(off, _CH)], ri_v)
            ca = pltpu.async_copy(p_hbm.at[si_v], ba, sa)
            cb = pltpu.async_copy(q_hbm.at[ri_v], bb, sb)
            ca.wait()
            cb.wait()
            pltpu.sync_copy(ba, so_hbm.at[pl.ds(off, _CH)])
            pltpu.sync_copy(bb, ro_hbm.at[pl.ds(off, _CH)])
            return carry

        lax.fori_loop(0, _NCH, body, 0)

    return k(p, q, s_idx, r_idx)


def _sc_scatter(upd, r_idx, zeros):
    mesh = plsc.VectorSubcoreMesh(core_axis_name="c", subcore_axis_name="s")

    @functools.partial(
        pl.kernel, mesh=mesh,
        out_type=jax.ShapeDtypeStruct((_NC, _NNP, _D), jnp.float32),
        scratch_types=[pltpu.VMEM((_CH,), jnp.int32),
                       pltpu.VMEM((_CH, _D), jnp.float32),
                       pltpu.VMEM_SHARED((_NNP, _D), jnp.float32)],
    )
    def k(u_hbm, r_hbm, z_hbm, o_hbm, ri_v, buf, agg_sh):
        c = lax.axis_index("c")
        s = lax.axis_index("s")
        pltpu.sync_copy(z_hbm.at[pl.ds(s * _RPT, _RPT)],
                        agg_sh.at[pl.ds(s * _RPT, _RPT)])
        plsc.subcore_barrier()
        base = c * _EPC + s * _EPW

        def body(i, carry):
            off = base + i * _CH
            pltpu.sync_copy(r_hbm.at[pl.ds(off, _CH)], ri_v)
            pltpu.sync_copy(u_hbm.at[pl.ds(off, _CH)], buf)
            pltpu.sync_copy(buf, agg_sh.at[ri_v], add=True)
            return carry

        lax.fori_loop(0, _NCH, body, 0)
        plsc.subcore_barrier()
        pltpu.sync_copy(agg_sh.at[pl.ds(s * _RPT, _RPT)],
                        o_hbm.at[c, pl.ds(s * _RPT, _RPT)])

    return k(upd, r_idx, zeros)


def kernel(node_features, mesh_edge_features, senders, receivers,
           edge_params, node_params):
    senders = senders.astype(jnp.int32)
    receivers = receivers.astype(jnp.int32)
    w1e = edge_params['w1']
    row = lambda v: v.reshape(1, _D)

    p, q = _premul(node_features, w1e[:2 * _D])
    s_feat, r_feat = _sc_gather(p, q, senders, receivers)
    upd, new_edge = _edge_mlp(
        s_feat, r_feat, mesh_edge_features, w1e[2 * _D:],
        row(edge_params['b1']), edge_params['w2'], row(edge_params['b2']),
        row(edge_params['ln_g']), row(edge_params['ln_b']))
    agg2 = _sc_scatter(upd, receivers,
                       jnp.zeros((_NNP, _D), jnp.float32))[:, :_NN, :]
    new_node = _node_mlp(
        node_features, agg2, node_params['w1'], row(node_params['b1']),
        node_params['w2'], row(node_params['b2']),
        row(node_params['ln_g']), row(node_params['ln_b']))
    return new_node, new_edge


# speedup vs baseline: 3.1161x; 1.2909x over previous
import functools

import jax
import jax.numpy as jnp
from jax import lax
from jax.experimental import pallas as pl
from jax.experimental.pallas import tpu as pltpu
from jax.experimental.pallas import tpu_sc as plsc

_NN = 10000
_NE = 320000
_D = 128
_CH = 80
_NC = 2
_NS = 16
_NW = _NC * _NS
_EPW = _NE // _NW
_NCH = _EPW // _CH
_EPC = _NE // _NC
_NNP = 10240
_RPT = _NNP // _NS


def _premul_body(n_ref, w_ref, p_ref, q_ref):
    n = n_ref[...]
    p_ref[...] = jnp.dot(n, w_ref[0:_D, :], preferred_element_type=jnp.float32)
    q_ref[...] = jnp.dot(n, w_ref[_D:2 * _D, :], preferred_element_type=jnp.float32)


def _premul(node, w1ab):
    b = 2000
    return pl.pallas_call(
        _premul_body,
        grid=(_NN // b,),
        in_specs=[pl.BlockSpec((b, _D), lambda i: (i, 0)),
                  pl.BlockSpec((2 * _D, _D), lambda i: (0, 0))],
        out_specs=[pl.BlockSpec((b, _D), lambda i: (i, 0)),
                   pl.BlockSpec((b, _D), lambda i: (i, 0))],
        out_shape=[jax.ShapeDtypeStruct((_NN, _D), jnp.float32),
                   jax.ShapeDtypeStruct((_NN, _D), jnp.float32)],
    )(node, w1ab)


def _edge_body(sr_ref, e_ref, w1c_ref, b1_ref, w2_ref, b2_ref,
               g_ref, bb_ref, u_ref, ne_ref):
    e = e_ref[...]
    x = (sr_ref[...] + b1_ref[...]
         + jnp.dot(e, w1c_ref[...], preferred_element_type=jnp.float32))
    h = jnp.maximum(x, 0.0)
    o = jnp.dot(h, w2_ref[...], preferred_element_type=jnp.float32) + b2_ref[...]
    mu = jnp.mean(o, axis=-1, keepdims=True)
    oc = o - mu
    var = jnp.mean(oc * oc, axis=-1, keepdims=True)
    u = oc * lax.rsqrt(var + 1e-5) * g_ref[...] + bb_ref[...]
    u_ref[...] = u
    ne_ref[...] = u + e


def _edge_mlp(sr_feat, e_feat, w1c, b1, w2, b2, ln_g, ln_b):
    b = 512
    row = lambda i: (i, 0)
    rep = lambda i: (0, 0)
    return pl.pallas_call(
        _edge_body,
        grid=(_NE // b,),
        in_specs=[pl.BlockSpec((b, _D), row),
                  pl.BlockSpec((b, _D), row),
                  pl.BlockSpec((_D, _D), rep),
                  pl.BlockSpec((1, _D), rep),
                  pl.BlockSpec((_D, _D), rep),
                  pl.BlockSpec((1, _D), rep),
                  pl.BlockSpec((1, _D), rep),
                  pl.BlockSpec((1, _D), rep)],
        out_specs=[pl.BlockSpec((b, _D), row),
                   pl.BlockSpec((b, _D), row)],
        out_shape=[jax.ShapeDtypeStruct((_NE, _D), jnp.float32),
                   jax.ShapeDtypeStruct((_NE, _D), jnp.float32)],
    )(sr_feat, e_feat, w1c, b1, w2, b2, ln_g, ln_b)


def _node_body(n_ref, a_ref, w1_ref, b1_ref, w2_ref, b2_ref, g_ref, bb_ref,
               o_ref):
    n = n_ref[...]
    a = a_ref[0, :, :] + a_ref[1, :, :]
    x = (jnp.dot(n, w1_ref[0:_D, :], preferred_element_type=jnp.float32)
         + jnp.dot(a, w1_ref[_D:2 * _D, :], preferred_element_type=jnp.float32)
         + b1_ref[...])
    h = jnp.maximum(x, 0.0)
    o = jnp.dot(h, w2_ref[...], preferred_element_type=jnp.float32) + b2_ref[...]
    mu = jnp.mean(o, axis=-1, keepdims=True)
    oc = o - mu
    var = jnp.mean(oc * oc, axis=-1, keepdims=True)
    o_ref[...] = oc * lax.rsqrt(var + 1e-5) * g_ref[...] + bb_ref[...] + n


def _node_mlp(node, agg2, w1, b1, w2, b2, ln_g, ln_b):
    b = 2000
    rep = lambda i: (0, 0)
    return pl.pallas_call(
        _node_body,
        grid=(_NN // b,),
        in_specs=[pl.BlockSpec((b, _D), lambda i: (i, 0)),
                  pl.BlockSpec((2, b, _D), lambda i: (0, i, 0)),
                  pl.BlockSpec((2 * _D, _D), rep),
                  pl.BlockSpec((1, _D), rep),
                  pl.BlockSpec((_D, _D), rep),
                  pl.BlockSpec((1, _D), rep),
                  pl.BlockSpec((1, _D), rep),
                  pl.BlockSpec((1, _D), rep)],
        out_specs=pl.BlockSpec((b, _D), lambda i: (i, 0)),
        out_shape=jax.ShapeDtypeStruct((_NN, _D), jnp.float32),
    )(node, agg2, w1, b1, w2, b2, ln_g, ln_b)


def _vadd_into(ba, bb):
    def vrow(r, carry):
        for j in range(_D // 16):
            sl = pl.ds(j * 16, 16)
            ba[r, sl] = ba[r, sl] + bb[r, sl]
        return carry
    lax.fori_loop(0, _CH, vrow, 0)


def _sc_gather(p, q, s_idx, r_idx):
    mesh = plsc.VectorSubcoreMesh(core_axis_name="c", subcore_axis_name="s")

    @functools.partial(
        pl.kernel, mesh=mesh,
        out_type=jax.ShapeDtypeStruct((_NE, _D), jnp.float32),
        scratch_types=[pltpu.VMEM((_CH,), jnp.int32),
                       pltpu.VMEM((_CH,), jnp.int32),
                       pltpu.VMEM((_CH,), jnp.int32),
                       pltpu.VMEM((_CH,), jnp.int32),
                       pltpu.VMEM((_CH, _D), jnp.float32),
                       pltpu.VMEM((_CH, _D), jnp.float32),
                       pltpu.VMEM((_CH, _D), jnp.float32),
                       pltpu.VMEM((_CH, _D), jnp.float32),
                       pltpu.SemaphoreType.DMA,
                       pltpu.SemaphoreType.DMA],
    )
    def k(p_hbm, q_hbm, s_hbm, r_hbm, g_hbm,
          si0, ri0, si1, ri1, ba0, bb0, ba1, bb1, sem0, sem1):
        wid = lax.axis_index("s") * _NC + lax.axis_index("c")
        base = wid * _EPW

        def start(chunk, si, ri, ba, bb, sem):
            off = base + chunk * _CH
            pltpu.sync_copy(s_hbm.at[pl.ds(off, _CH)], si)
            pltpu.sync_copy(r_hbm.at[pl.ds(off, _CH)], ri)
            pltpu.async_copy(p_hbm.at[si], ba, sem)
            pltpu.async_copy(q_hbm.at[ri], bb, sem)

        def finish(chunk, si, ri, ba, bb, sem):
            pltpu.make_async_copy(p_hbm.at[si], ba, sem).wait()
            pltpu.make_async_copy(q_hbm.at[ri], bb, sem).wait()
            _vadd_into(ba, bb)
            pltpu.sync_copy(ba, g_hbm.at[pl.ds(base + chunk * _CH, _CH)])

        start(0, si0, ri0, ba0, bb0, sem0)

        def body(kk, carry):
            c0 = 2 * kk
            start(c0 + 1, si1, ri1, ba1, bb1, sem1)
            finish(c0, si0, ri0, ba0, bb0, sem0)
            start(c0 + 2, si0, ri0, ba0, bb0, sem0)
            finish(c0 + 1, si1, ri1, ba1, bb1, sem1)
            return carry

        lax.fori_loop(0, (_NCH - 1) // 2, body, 0)
        finish(_NCH - 1, si0, ri0, ba0, bb0, sem0)

    return k(p, q, s_idx, r_idx)


def _sc_scatter(upd, r_idx, zeros):
    mesh = plsc.VectorSubcoreMesh(core_axis_name="c", subcore_axis_name="s")

    @functools.partial(
        pl.kernel, mesh=mesh,
        out_type=jax.ShapeDtypeStruct((_NC, _NNP, _D), jnp.float32),
        scratch_types=[pltpu.VMEM((_CH,), jnp.int32),
                       pltpu.VMEM((_CH,), jnp.int32),
                       pltpu.VMEM((_CH, _D), jnp.float32),
                       pltpu.VMEM((_CH, _D), jnp.float32),
                       pltpu.SemaphoreType.DMA,
                       pltpu.SemaphoreType.DMA,
                       pltpu.VMEM_SHARED((_NNP, _D), jnp.float32)],
    )
    def k(u_hbm, r_hbm, z_hbm, o_hbm, ri0, ri1, buf0, buf1, sem0, sem1,
          agg_sh):
        c = lax.axis_index("c")
        s = lax.axis_index("s")
        pltpu.sync_copy(z_hbm.at[pl.ds(s * _RPT, _RPT)],
                        agg_sh.at[pl.ds(s * _RPT, _RPT)])
        plsc.subcore_barrier()
        base = c * _EPC + s * _EPW

        def start(chunk, ri, buf, sem):
            off = base + chunk * _CH
            pltpu.sync_copy(r_hbm.at[pl.ds(off, _CH)], ri)
            pltpu.async_copy(u_hbm.at[pl.ds(off, _CH)], buf, sem)

        def finish(ri, buf, sem):
            pltpu.make_async_copy(u_hbm.at[pl.ds(base, _CH)], buf, sem).wait()
            pltpu.sync_copy(buf, agg_sh.at[ri], add=True)

        start(0, ri0, buf0, sem0)

        def body(kk, carry):
            c0 = 2 * kk
            start(c0 + 1, ri1, buf1, sem1)
            finish(ri0, buf0, sem0)
            start(c0 + 2, ri0, buf0, sem0)
            finish(ri1, buf1, sem1)
            return carry

        lax.fori_loop(0, (_NCH - 1) // 2, body, 0)
        finish(ri0, buf0, sem0)
        plsc.subcore_barrier()
        pltpu.sync_copy(agg_sh.at[pl.ds(s * _RPT, _RPT)],
                        o_hbm.at[c, pl.ds(s * _RPT, _RPT)])

    return k(upd, r_idx, zeros)


def kernel(node_features, mesh_edge_features, senders, receivers,
           edge_params, node_params):
    senders = senders.astype(jnp.int32)
    receivers = receivers.astype(jnp.int32)
    w1e = edge_params['w1']
    row = lambda v: v.reshape(1, _D)

    p, q = _premul(node_features, w1e[:2 * _D])
    sr_feat = _sc_gather(p, q, senders, receivers)
    upd, new_edge = _edge_mlp(
        sr_feat, mesh_edge_features, w1e[2 * _D:],
        row(edge_params['b1']), edge_params['w2'], row(edge_params['b2']),
        row(edge_params['ln_g']), row(edge_params['ln_b']))
    agg2 = _sc_scatter(upd, receivers,
                       jnp.zeros((_NNP, _D), jnp.float32))[:, :_NN, :]
    new_node = _node_mlp(
        node_features, agg2, node_params['w1'], row(node_params['b1']),
        node_params['w2'], row(node_params['b2']),
        row(node_params['ln_g']), row(node_params['ln_b']))
    return new_node, new_edge


# speedup vs baseline: 3.6873x; 1.1833x over previous
import functools

import jax
import jax.numpy as jnp
from jax import lax
from jax.experimental import pallas as pl
from jax.experimental.pallas import tpu as pltpu
from jax.experimental.pallas import tpu_sc as plsc

_NN = 10000
_NE = 320000
_D = 128
_CH = 80
_NC = 2
_NS = 16
_NW = _NC * _NS
_K = 5
_SL = _NE // _K
_EPWS = _SL // _NW
_NCHS = _EPWS // _CH
_EB = 512
_NBS = _SL // _EB
_NNP = 10240
_RPT = _NNP // _NS


def _premul_body(n_ref, w_ref, p_ref, q_ref):
    n = n_ref[...]
    p_ref[...] = jnp.dot(n, w_ref[0:_D, :], preferred_element_type=jnp.float32)
    q_ref[...] = jnp.dot(n, w_ref[_D:2 * _D, :], preferred_element_type=jnp.float32)


def _premul(node, w1ab):
    b = 2000
    return pl.pallas_call(
        _premul_body,
        grid=(_NN // b,),
        in_specs=[pl.BlockSpec((b, _D), lambda i: (i, 0)),
                  pl.BlockSpec((2 * _D, _D), lambda i: (0, 0))],
        out_specs=[pl.BlockSpec((b, _D), lambda i: (i, 0)),
                   pl.BlockSpec((b, _D), lambda i: (i, 0))],
        out_shape=[jax.ShapeDtypeStruct((_NN, _D), jnp.float32),
                   jax.ShapeDtypeStruct((_NN, _D), jnp.float32)],
    )(node, w1ab)


def _edge_body(g_ref, e_ref, w1c_ref, b1_ref, w2_ref, b2_ref,
               lg_ref, lb_ref, *rest):
    u_ref, ne_ref = rest[-2], rest[-1]
    e = e_ref[...]
    x = (g_ref[...] + b1_ref[...]
         + jnp.dot(e, w1c_ref[...], preferred_element_type=jnp.float32))
    h = jnp.maximum(x, 0.0)
    o = jnp.dot(h, w2_ref[...], preferred_element_type=jnp.float32) + b2_ref[...]
    mu = jnp.mean(o, axis=-1, keepdims=True)
    oc = o - mu
    var = jnp.mean(oc * oc, axis=-1, keepdims=True)
    u = oc * lax.rsqrt(var + 1e-5) * lg_ref[...] + lb_ref[...]
    u_ref[...] = u
    ne_ref[...] = u + e


def _edge_mlp_slice(g, e_feat, w1c, b1, w2, b2, ln_g, ln_b, ks, ne_alias):
    base = ks * _NBS
    row_l = lambda i: (i, 0)
    row_g = lambda i, base=base: (i + base, 0)
    rep = lambda i: (0, 0)
    ins = [g, e_feat, w1c, b1, w2, b2, ln_g, ln_b]
    in_specs = [pl.BlockSpec((_EB, _D), row_l),
                pl.BlockSpec((_EB, _D), row_g),
                pl.BlockSpec((_D, _D), rep),
                pl.BlockSpec((1, _D), rep),
                pl.BlockSpec((_D, _D), rep),
                pl.BlockSpec((1, _D), rep),
                pl.BlockSpec((1, _D), rep),
                pl.BlockSpec((1, _D), rep)]
    aliases = {}
    if ne_alias is not None:
        ins.append(ne_alias)
        in_specs.append(pl.BlockSpec(memory_space=pl.ANY))
        aliases = {8: 1}
    return pl.pallas_call(
        _edge_body,
        grid=(_NBS,),
        in_specs=in_specs,
        out_specs=[pl.BlockSpec((_EB, _D), row_l),
                   pl.BlockSpec((_EB, _D), row_g)],
        out_shape=[jax.ShapeDtypeStruct((_SL, _D), jnp.float32),
                   jax.ShapeDtypeStruct((_NE, _D), jnp.float32)],
        input_output_aliases=aliases,
    )(*ins)


def _node_body(n_ref, a_ref, w1_ref, b1_ref, w2_ref, b2_ref, lg_ref, lb_ref,
               o_ref):
    n = n_ref[...]
    a = a_ref[0, :, :] + a_ref[1, :, :]
    x = (jnp.dot(n, w1_ref[0:_D, :], preferred_element_type=jnp.float32)
         + jnp.dot(a, w1_ref[_D:2 * _D, :], preferred_element_type=jnp.float32)
         + b1_ref[...])
    h = jnp.maximum(x, 0.0)
    o = jnp.dot(h, w2_ref[...], preferred_element_type=jnp.float32) + b2_ref[...]
    mu = jnp.mean(o, axis=-1, keepdims=True)
    oc = o - mu
    var = jnp.mean(oc * oc, axis=-1, keepdims=True)
    o_ref[...] = oc * lax.rsqrt(var + 1e-5) * lg_ref[...] + lb_ref[...] + n


def _node_mlp(node, agg2, w1, b1, w2, b2, ln_g, ln_b):
    b = 2000
    rep = lambda i: (0, 0)
    return pl.pallas_call(
        _node_body,
        grid=(_NN // b,),
        in_specs=[pl.BlockSpec((b, _D), lambda i: (i, 0)),
                  pl.BlockSpec((2, b, _D), lambda i: (0, i, 0)),
                  pl.BlockSpec((2 * _D, _D), rep),
                  pl.BlockSpec((1, _D), rep),
                  pl.BlockSpec((_D, _D), rep),
                  pl.BlockSpec((1, _D), rep),
                  pl.BlockSpec((1, _D), rep),
                  pl.BlockSpec((1, _D), rep)],
        out_specs=pl.BlockSpec((b, _D), lambda i: (i, 0)),
        out_shape=jax.ShapeDtypeStruct((_NN, _D), jnp.float32),
    )(node, agg2, w1, b1, w2, b2, ln_g, ln_b)


def _vadd_into(ba, bb):
    def vrow(r, carry):
        for j in range(_D // 16):
            sl = pl.ds(j * 16, 16)
            ba[r, sl] = ba[r, sl] + bb[r, sl]
        return carry
    lax.fori_loop(0, _CH, vrow, 0)


def _sc_gather_slice(p, q, s_idx, r_idx, ks):
    mesh = plsc.VectorSubcoreMesh(core_axis_name="c", subcore_axis_name="s")

    @functools.partial(
        pl.kernel, mesh=mesh,
        out_type=jax.ShapeDtypeStruct((_SL, _D), jnp.float32),
        scratch_types=[pltpu.VMEM((_CH,), jnp.int32),
                       pltpu.VMEM((_CH,), jnp.int32),
                       pltpu.VMEM((_CH,), jnp.int32),
                       pltpu.VMEM((_CH,), jnp.int32),
                       pltpu.VMEM((_CH, _D), jnp.float32),
                       pltpu.VMEM((_CH, _D), jnp.float32),
                       pltpu.VMEM((_CH, _D), jnp.float32),
                       pltpu.VMEM((_CH, _D), jnp.float32),
                       pltpu.SemaphoreType.DMA,
                       pltpu.SemaphoreType.DMA],
    )
    def k(p_hbm, q_hbm, s_hbm, r_hbm, g_hbm,
          si0, ri0, si1, ri1, ba0, bb0, ba1, bb1, sem0, sem1):
        wid = lax.axis_index("s") * _NC + lax.axis_index("c")
        ibase = ks * _SL + wid * _EPWS
        obase = wid * _EPWS

        def start(chunk, si, ri, ba, bb, sem):
            off = ibase + chunk * _CH
            pltpu.sync_copy(s_hbm.at[pl.ds(off, _CH)], si)
            pltpu.sync_copy(r_hbm.at[pl.ds(off, _CH)], ri)
            pltpu.async_copy(p_hbm.at[si], ba, sem)
            pltpu.async_copy(q_hbm.at[ri], bb, sem)

        def finish(chunk, si, ri, ba, bb, sem):
            pltpu.make_async_copy(p_hbm.at[si], ba, sem).wait()
            pltpu.make_async_copy(q_hbm.at[ri], bb, sem).wait()
            _vadd_into(ba, bb)
            pltpu.sync_copy(ba, g_hbm.at[pl.ds(obase + chunk * _CH, _CH)])

        start(0, si0, ri0, ba0, bb0, sem0)

        def body(kk, carry):
            c0 = 2 * kk
            start(c0 + 1, si1, ri1, ba1, bb1, sem1)
            finish(c0, si0, ri0, ba0, bb0, sem0)
            start(c0 + 2, si0, ri0, ba0, bb0, sem0)
            finish(c0 + 1, si1, ri1, ba1, bb1, sem1)
            return carry

        lax.fori_loop(0, (_NCHS - 1) // 2, body, 0)
        finish(_NCHS - 1, si0, ri0, ba0, bb0, sem0)

    return k(p, q, s_idx, r_idx)


def _sc_scatter(upds, r_idx):
    mesh = plsc.VectorSubcoreMesh(core_axis_name="c", subcore_axis_name="s")

    @functools.partial(
        pl.kernel, mesh=mesh,
        out_type=jax.ShapeDtypeStruct((_NC, _NNP, _D), jnp.float32),
        scratch_types=[pltpu.VMEM((_CH,), jnp.int32),
                       pltpu.VMEM((_CH,), jnp.int32),
                       pltpu.VMEM((_CH, _D), jnp.float32),
                       pltpu.VMEM((_CH, _D), jnp.float32),
                       pltpu.SemaphoreType.DMA,
                       pltpu.SemaphoreType.DMA,
                       pltpu.VMEM_SHARED((_NNP, _D), jnp.float32)],
    )
    def k(u0, u1, u2, u3, u4, r_hbm, o_hbm, ri0, ri1, buf0, buf1,
          sem0, sem1, agg_sh):
        c = lax.axis_index("c")
        s = lax.axis_index("s")
        wid = s * _NC + c

        def zrow(r, carry):
            for j in range(_D // 16):
                buf0[r, pl.ds(j * 16, 16)] = jnp.zeros((16,), jnp.float32)
            return carry
        lax.fori_loop(0, _CH, zrow, 0)
        for t in range(_RPT // _CH):
            pltpu.sync_copy(buf0, agg_sh.at[pl.ds(s * _RPT + t * _CH, _CH)])
        plsc.subcore_barrier()

        for ks, u_hbm in enumerate((u0, u1, u2, u3, u4)):
            ibase = ks * _SL + wid * _EPWS
            ubase = wid * _EPWS

            def start(chunk, ri, buf, sem):
                pltpu.sync_copy(r_hbm.at[pl.ds(ibase + chunk * _CH, _CH)], ri)
                pltpu.async_copy(u_hbm.at[pl.ds(ubase + chunk * _CH, _CH)],
                                 buf, sem)

            def finish(ri, buf, sem):
                pltpu.make_async_copy(u_hbm.at[pl.ds(ubase, _CH)],
                                      buf, sem).wait()
                pltpu.sync_copy(buf, agg_sh.at[ri], add=True)

            start(0, ri0, buf0, sem0)

            def body(kk, carry):
                c0 = 2 * kk
                start(c0 + 1, ri1, buf1, sem1)
                finish(ri0, buf0, sem0)
                start(c0 + 2, ri0, buf0, sem0)
                finish(ri1, buf1, sem1)
                return carry

            lax.fori_loop(0, (_NCHS - 1) // 2, body, 0)
            finish(ri0, buf0, sem0)

        plsc.subcore_barrier()
        pltpu.sync_copy(agg_sh.at[pl.ds(s * _RPT, _RPT)],
                        o_hbm.at[c, pl.ds(s * _RPT, _RPT)])

    return k(*upds, r_idx)


def kernel(node_features, mesh_edge_features, senders, receivers,
           edge_params, node_params):
    senders = senders.astype(jnp.int32)
    receivers = receivers.astype(jnp.int32)
    w1e = edge_params['w1']
    row = lambda v: v.reshape(1, _D)

    p, q = _premul(node_features, w1e[:2 * _D])

    upds = []
    new_edge = None
    for ks in range(_K):
        g = _sc_gather_slice(p, q, senders, receivers, ks)
        upd_k, new_edge = _edge_mlp_slice(
            g, mesh_edge_features, w1e[2 * _D:],
            row(edge_params['b1']), edge_params['w2'], row(edge_params['b2']),
            row(edge_params['ln_g']), row(edge_params['ln_b']),
            ks, new_edge)
        upds.append(upd_k)

    agg2 = _sc_scatter(upds, receivers)
    new_node = _node_mlp(
        node_features, agg2, node_params['w1'], row(node_params['b1']),
        node_params['w2'], row(node_params['b2']),
        row(node_params['ln_g']), row(node_params['ln_b']))
    return new_node, new_edge


# speedup vs baseline: 4.0191x; 1.0900x over previous
import functools

import jax
import jax.numpy as jnp
from jax import lax
from jax.experimental import pallas as pl
from jax.experimental.pallas import tpu as pltpu
from jax.experimental.pallas import tpu_sc as plsc

_NN = 10000
_NE = 320000
_D = 128
_CH = 80
_NC = 2
_NS = 16
_NW = _NC * _NS
_K = 5
_SL = _NE // _K
_EPWS = _SL // _NW
_NCHS = _EPWS // _CH
_EB = 512
_NBS = _SL // _EB
_NNP = 10240
_RPT = _NNP // _NS


def _premul_body(n_ref, w_ref, p_ref, q_ref):
    n = n_ref[...]
    p_ref[...] = jnp.dot(n, w_ref[0:_D, :], preferred_element_type=jnp.float32)
    q_ref[...] = jnp.dot(n, w_ref[_D:2 * _D, :], preferred_element_type=jnp.float32)


def _premul(node, w1ab):
    b = 2000
    return pl.pallas_call(
        _premul_body,
        grid=(_NN // b,),
        in_specs=[pl.BlockSpec((b, _D), lambda i: (i, 0)),
                  pl.BlockSpec((2 * _D, _D), lambda i: (0, 0))],
        out_specs=[pl.BlockSpec((b, _D), lambda i: (i, 0)),
                   pl.BlockSpec((b, _D), lambda i: (i, 0))],
        out_shape=[jax.ShapeDtypeStruct((_NN, _D), jnp.float32),
                   jax.ShapeDtypeStruct((_NN, _D), jnp.float32)],
    )(node, w1ab)


def _edge_body(g_ref, e_ref, w1c_ref, b1_ref, w2_ref, b2_ref,
               lg_ref, lb_ref, *rest):
    u_ref, ne_ref = rest[-2], rest[-1]
    e = e_ref[...]
    x = (g_ref[...] + b1_ref[...]
         + jnp.dot(e, w1c_ref[...], preferred_element_type=jnp.float32))
    h = jnp.maximum(x, 0.0)
    o = jnp.dot(h, w2_ref[...], preferred_element_type=jnp.float32) + b2_ref[...]
    mu = jnp.mean(o, axis=-1, keepdims=True)
    oc = o - mu
    var = jnp.mean(oc * oc, axis=-1, keepdims=True)
    u = oc * lax.rsqrt(var + 1e-5) * lg_ref[...] + lb_ref[...]
    u_ref[...] = u
    ne_ref[...] = u + e


def _edge_mlp_slice(g, e_feat, w1c, b1, w2, b2, ln_g, ln_b, ks, ne_alias):
    base = ks * _NBS
    row_l = lambda i: (i, 0)
    row_g = lambda i, base=base: (i + base, 0)
    rep = lambda i: (0, 0)
    ins = [g, e_feat, w1c, b1, w2, b2, ln_g, ln_b]
    in_specs = [pl.BlockSpec((_EB, _D), row_l),
                pl.BlockSpec((_EB, _D), row_g),
                pl.BlockSpec((_D, _D), rep),
                pl.BlockSpec((1, _D), rep),
                pl.BlockSpec((_D, _D), rep),
                pl.BlockSpec((1, _D), rep),
                pl.BlockSpec((1, _D), rep),
                pl.BlockSpec((1, _D), rep)]
    aliases = {}
    if ne_alias is not None:
        ins.append(ne_alias)
        in_specs.append(pl.BlockSpec(memory_space=pl.ANY))
        aliases = {8: 1}
    return pl.pallas_call(
        _edge_body,
        grid=(_NBS,),
        in_specs=in_specs,
        out_specs=[pl.BlockSpec((_EB, _D), row_l),
                   pl.BlockSpec((_EB, _D), row_g)],
        out_shape=[jax.ShapeDtypeStruct((_SL, _D), jnp.float32),
                   jax.ShapeDtypeStruct((_NE, _D), jnp.float32)],
        input_output_aliases=aliases,
    )(*ins)


def _node_body(n_ref, a_ref, w1_ref, b1_ref, w2_ref, b2_ref, lg_ref, lb_ref,
               o_ref):
    n = n_ref[...]
    a = a_ref[0, :, :] + a_ref[1, :, :]
    x = (jnp.dot(n, w1_ref[0:_D, :], preferred_element_type=jnp.float32)
         + jnp.dot(a, w1_ref[_D:2 * _D, :], preferred_element_type=jnp.float32)
         + b1_ref[...])
    h = jnp.maximum(x, 0.0)
    o = jnp.dot(h, w2_ref[...], preferred_element_type=jnp.float32) + b2_ref[...]
    mu = jnp.mean(o, axis=-1, keepdims=True)
    oc = o - mu
    var = jnp.mean(oc * oc, axis=-1, keepdims=True)
    o_ref[...] = oc * lax.rsqrt(var + 1e-5) * lg_ref[...] + lb_ref[...] + n


def _node_mlp(node, agg2, w1, b1, w2, b2, ln_g, ln_b):
    b = 2000
    rep = lambda i: (0, 0)
    return pl.pallas_call(
        _node_body,
        grid=(_NN // b,),
        in_specs=[pl.BlockSpec((b, _D), lambda i: (i, 0)),
                  pl.BlockSpec((2, b, _D), lambda i: (0, i, 0)),
                  pl.BlockSpec((2 * _D, _D), rep),
                  pl.BlockSpec((1, _D), rep),
                  pl.BlockSpec((_D, _D), rep),
                  pl.BlockSpec((1, _D), rep),
                  pl.BlockSpec((1, _D), rep),
                  pl.BlockSpec((1, _D), rep)],
        out_specs=pl.BlockSpec((b, _D), lambda i: (i, 0)),
        out_shape=jax.ShapeDtypeStruct((_NN, _D), jnp.float32),
    )(node, agg2, w1, b1, w2, b2, ln_g, ln_b)


def _vadd_into(ba, bb):
    def vrow(r, carry):
        for j in range(_D // 16):
            sl = pl.ds(j * 16, 16)
            ba[r, sl] = ba[r, sl] + bb[r, sl]
        return carry
    lax.fori_loop(0, _CH, vrow, 0)


def _sc_gather_slice(p, q, s_idx, r_idx, ks):
    mesh = plsc.VectorSubcoreMesh(core_axis_name="c", subcore_axis_name="s")

    @functools.partial(
        pl.kernel, mesh=mesh,
        out_type=jax.ShapeDtypeStruct((_SL, _D), jnp.float32),
        scratch_types=[pltpu.VMEM((_CH,), jnp.int32),
                       pltpu.VMEM((_CH,), jnp.int32),
                       pltpu.VMEM((_CH,), jnp.int32),
                       pltpu.VMEM((_CH,), jnp.int32),
                       pltpu.VMEM((_CH, _D), jnp.float32),
                       pltpu.VMEM((_CH, _D), jnp.float32),
                       pltpu.VMEM((_CH, _D), jnp.float32),
                       pltpu.VMEM((_CH, _D), jnp.float32),
                       pltpu.SemaphoreType.DMA,
                       pltpu.SemaphoreType.DMA],
    )
    def k(p_hbm, q_hbm, s_hbm, r_hbm, g_hbm,
          si0, ri0, si1, ri1, ba0, bb0, ba1, bb1, sem0, sem1):
        wid = lax.axis_index("s") * _NC + lax.axis_index("c")
        ibase = ks * _SL + wid * _EPWS
        obase = wid * _EPWS

        def start(chunk, si, ri, ba, bb, sem):
            off = ibase + chunk * _CH
            pltpu.sync_copy(s_hbm.at[pl.ds(off, _CH)], si)
            pltpu.sync_copy(r_hbm.at[pl.ds(off, _CH)], ri)
            pltpu.async_copy(p_hbm.at[si], ba, sem)
            pltpu.async_copy(q_hbm.at[ri], bb, sem)

        def finish(chunk, si, ri, ba, bb, sem):
            pltpu.make_async_copy(p_hbm.at[si], ba, sem).wait()
            pltpu.make_async_copy(q_hbm.at[ri], bb, sem).wait()
            _vadd_into(ba, bb)
            pltpu.sync_copy(ba, g_hbm.at[pl.ds(obase + chunk * _CH, _CH)])

        start(0, si0, ri0, ba0, bb0, sem0)

        def body(kk, carry):
            c0 = 2 * kk
            start(c0 + 1, si1, ri1, ba1, bb1, sem1)
            finish(c0, si0, ri0, ba0, bb0, sem0)
            start(c0 + 2, si0, ri0, ba0, bb0, sem0)
            finish(c0 + 1, si1, ri1, ba1, bb1, sem1)
            return carry

        lax.fori_loop(0, (_NCHS - 1) // 2, body, 0)
        finish(_NCHS - 1, si0, ri0, ba0, bb0, sem0)

    return k(p, q, s_idx, r_idx)


def _sc_scatter(upds, r_idx, ks0):
    mesh = plsc.VectorSubcoreMesh(core_axis_name="c", subcore_axis_name="s")

    @functools.partial(
        pl.kernel, mesh=mesh,
        out_type=jax.ShapeDtypeStruct((_NC, _NNP, _D), jnp.float32),
        scratch_types=[pltpu.VMEM((_CH,), jnp.int32),
                       pltpu.VMEM((_CH,), jnp.int32),
                       pltpu.VMEM((_CH, _D), jnp.float32),
                       pltpu.VMEM((_CH, _D), jnp.float32),
                       pltpu.SemaphoreType.DMA,
                       pltpu.SemaphoreType.DMA,
                       pltpu.VMEM_SHARED((_NNP, _D), jnp.float32)],
    )
    def k(*refs):
        u_hbms = refs[:len(upds)]
        (r_hbm, o_hbm, ri0, ri1, buf0, buf1,
         sem0, sem1, agg_sh) = refs[len(upds):]
        c = lax.axis_index("c")
        s = lax.axis_index("s")
        wid = s * _NC + c

        def zrow(r, carry):
            for j in range(_D // 16):
                buf0[r, pl.ds(j * 16, 16)] = jnp.zeros((16,), jnp.float32)
            return carry
        lax.fori_loop(0, _CH, zrow, 0)
        for t in range(_RPT // _CH):
            pltpu.sync_copy(buf0, agg_sh.at[pl.ds(s * _RPT + t * _CH, _CH)])
        plsc.subcore_barrier()

        for ku, u_hbm in enumerate(u_hbms):
            ibase = (ks0 + ku) * _SL + wid * _EPWS
            ubase = wid * _EPWS

            def start(chunk, ri, buf, sem):
                pltpu.sync_copy(r_hbm.at[pl.ds(ibase + chunk * _CH, _CH)], ri)
                pltpu.async_copy(u_hbm.at[pl.ds(ubase + chunk * _CH, _CH)],
                                 buf, sem)

            def finish(ri, buf, sem):
                pltpu.make_async_copy(u_hbm.at[pl.ds(ubase, _CH)],
                                      buf, sem).wait()
                pltpu.sync_copy(buf, agg_sh.at[ri], add=True)

            start(0, ri0, buf0, sem0)

            def body(kk, carry):
                c0 = 2 * kk
                start(c0 + 1, ri1, buf1, sem1)
                finish(ri0, buf0, sem0)
                start(c0 + 2, ri0, buf0, sem0)
                finish(ri1, buf1, sem1)
                return carry

            lax.fori_loop(0, (_NCHS - 1) // 2, body, 0)
            finish(ri0, buf0, sem0)

        plsc.subcore_barrier()
        pltpu.sync_copy(agg_sh.at[pl.ds(s * _RPT, _RPT)],
                        o_hbm.at[c, pl.ds(s * _RPT, _RPT)])

    return k(*upds, r_idx)


def _node_body4(n_ref, a_ref, b_ref, w1_ref, b1_ref, w2_ref, b2_ref,
                lg_ref, lb_ref, o_ref):
    n = n_ref[...]
    a = (a_ref[0, :, :] + a_ref[1, :, :]
         + b_ref[0, :, :] + b_ref[1, :, :])
    x = (jnp.dot(n, w1_ref[0:_D, :], preferred_element_type=jnp.float32)
         + jnp.dot(a, w1_ref[_D:2 * _D, :], preferred_element_type=jnp.float32)
         + b1_ref[...])
    h = jnp.maximum(x, 0.0)
    o = jnp.dot(h, w2_ref[...], preferred_element_type=jnp.float32) + b2_ref[...]
    mu = jnp.mean(o, axis=-1, keepdims=True)
    oc = o - mu
    var = jnp.mean(oc * oc, axis=-1, keepdims=True)
    o_ref[...] = oc * lax.rsqrt(var + 1e-5) * lg_ref[...] + lb_ref[...] + n


def _node_mlp4(node, agg_a, agg_b, w1, b1, w2, b2, ln_g, ln_b):
    b = 2000
    rep = lambda i: (0, 0)
    agg_spec = pl.BlockSpec((2, b, _D), lambda i: (0, i, 0))
    return pl.pallas_call(
        _node_body4,
        grid=(_NN // b,),
        in_specs=[pl.BlockSpec((b, _D), lambda i: (i, 0)),
                  agg_spec, agg_spec,
                  pl.BlockSpec((2 * _D, _D), rep),
                  pl.BlockSpec((1, _D), rep),
                  pl.BlockSpec((_D, _D), rep),
                  pl.BlockSpec((1, _D), rep),
                  pl.BlockSpec((1, _D), rep),
                  pl.BlockSpec((1, _D), rep)],
        out_specs=pl.BlockSpec((b, _D), lambda i: (i, 0)),
        out_shape=jax.ShapeDtypeStruct((_NN, _D), jnp.float32),
    )(node, agg_a, agg_b, w1, b1, w2, b2, ln_g, ln_b)


def kernel(node_features, mesh_edge_features, senders, receivers,
           edge_params, node_params):
    senders = senders.astype(jnp.int32)
    receivers = receivers.astype(jnp.int32)
    w1e = edge_params['w1']
    row = lambda v: v.reshape(1, _D)

    p, q = _premul(node_features, w1e[:2 * _D])

    upds = []
    new_edge = None
    for ks in range(_K):
        g = _sc_gather_slice(p, q, senders, receivers, ks)
        upd_k, new_edge = _edge_mlp_slice(
            g, mesh_edge_features, w1e[2 * _D:],
            row(edge_params['b1']), edge_params['w2'], row(edge_params['b2']),
            row(edge_params['ln_g']), row(edge_params['ln_b']),
            ks, new_edge)
        upds.append(upd_k)

    agg_a = _sc_scatter(upds[:3], receivers, 0)
    agg_b = _sc_scatter(upds[3:], receivers, 3)
    new_node = _node_mlp4(
        node_features, agg_a, agg_b, node_params['w1'], row(node_params['b1']),
        node_params['w2'], row(node_params['b2']),
        row(node_params['ln_g']), row(node_params['ln_b']))
    return new_node, new_edge


# speedup vs baseline: 4.1313x; 1.0279x over previous
import functools

import jax
import jax.numpy as jnp
import numpy as np
from jax import lax
from jax.experimental import pallas as pl
from jax.experimental.pallas import tpu as pltpu
from jax.experimental.pallas import tpu_sc as plsc

_NN = 10000
_NE = 320000
_D = 128
_CH = 80
_NC = 2
_NS = 16
_NW = _NC * _NS
_K = 5
_SL = _NE // _K
_EPWS = _SL // _NW
_NCHS = _EPWS // _CH
_EB = 512
_NBS = _SL // _EB
_NNP = 10240
_RPT = _NNP // _NS

_PERM = np.empty((_D,), np.int32)
for _j in range(_D // 32):
    for _k in range(16):
        _PERM[32 * _j + 2 * _k] = 32 * _j + _k
        _PERM[32 * _j + 2 * _k + 1] = 32 * _j + 16 + _k


def _premul_body(n_ref, w_ref, p_ref, q_ref):
    n = n_ref[...]
    p_ref[...] = jnp.dot(n, w_ref[0:_D, :], preferred_element_type=jnp.float32)
    q_ref[...] = jnp.dot(n, w_ref[_D:2 * _D, :], preferred_element_type=jnp.float32)


def _premul(node, w1ab):
    b = 2000
    return pl.pallas_call(
        _premul_body,
        grid=(_NN // b,),
        in_specs=[pl.BlockSpec((b, _D), lambda i: (i, 0)),
                  pl.BlockSpec((2 * _D, _D), lambda i: (0, 0))],
        out_specs=[pl.BlockSpec((b, _D), lambda i: (i, 0)),
                   pl.BlockSpec((b, _D), lambda i: (i, 0))],
        out_shape=[jax.ShapeDtypeStruct((_NN, _D), jnp.float32),
                   jax.ShapeDtypeStruct((_NN, _D), jnp.float32)],
    )(node, w1ab)


def _edge_body(g_ref, e_ref, w1c_ref, b1_ref, w2_ref, b2_ref,
               lg_ref, lb_ref, *rest):
    u_ref, ne_ref = rest[-2], rest[-1]
    e = e_ref[...]
    x = (g_ref[...].astype(jnp.float32) + b1_ref[...]
         + jnp.dot(e, w1c_ref[...], preferred_element_type=jnp.float32))
    h = jnp.maximum(x, 0.0)
    o = jnp.dot(h, w2_ref[...], preferred_element_type=jnp.float32) + b2_ref[...]
    mu = jnp.mean(o, axis=-1, keepdims=True)
    oc = o - mu
    var = jnp.mean(oc * oc, axis=-1, keepdims=True)
    u = oc * lax.rsqrt(var + 1e-5) * lg_ref[...] + lb_ref[...]
    u_ref[...] = u
    ne_ref[...] = u + e


def _edge_mlp_slice(g, e_feat, w1c, b1, w2, b2, ln_g, ln_b, ks, ne_alias):
    base = ks * _NBS
    row_l = lambda i: (i, 0)
    row_g = lambda i, base=base: (i + base, 0)
    rep = lambda i: (0, 0)
    ins = [g, e_feat, w1c, b1, w2, b2, ln_g, ln_b]
    in_specs = [pl.BlockSpec((_EB, _D), row_l),
                pl.BlockSpec((_EB, _D), row_g),
                pl.BlockSpec((_D, _D), rep),
                pl.BlockSpec((1, _D), rep),
                pl.BlockSpec((_D, _D), rep),
                pl.BlockSpec((1, _D), rep),
                pl.BlockSpec((1, _D), rep),
                pl.BlockSpec((1, _D), rep)]
    aliases = {}
    if ne_alias is not None:
        ins.append(ne_alias)
        in_specs.append(pl.BlockSpec(memory_space=pl.ANY))
        aliases = {8: 1}
    return pl.pallas_call(
        _edge_body,
        grid=(_NBS,),
        in_specs=in_specs,
        out_specs=[pl.BlockSpec((_EB, _D), row_l),
                   pl.BlockSpec((_EB, _D), row_g)],
        out_shape=[jax.ShapeDtypeStruct((_SL, _D), jnp.float32),
                   jax.ShapeDtypeStruct((_NE, _D), jnp.float32)],
        input_output_aliases=aliases,
    )(*ins)


def _node_body(n_ref, a_ref, w1_ref, b1_ref, w2_ref, b2_ref, lg_ref, lb_ref,
               o_ref):
    n = n_ref[...]
    a = a_ref[0, :, :] + a_ref[1, :, :]
    x = (jnp.dot(n, w1_ref[0:_D, :], preferred_element_type=jnp.float32)
         + jnp.dot(a, w1_ref[_D:2 * _D, :], preferred_element_type=jnp.float32)
         + b1_ref[...])
    h = jnp.maximum(x, 0.0)
    o = jnp.dot(h, w2_ref[...], preferred_element_type=jnp.float32) + b2_ref[...]
    mu = jnp.mean(o, axis=-1, keepdims=True)
    oc = o - mu
    var = jnp.mean(oc * oc, axis=-1, keepdims=True)
    o_ref[...] = oc * lax.rsqrt(var + 1e-5) * lg_ref[...] + lb_ref[...] + n


def _node_mlp(node, agg2, w1, b1, w2, b2, ln_g, ln_b):
    b = 2000
    rep = lambda i: (0, 0)
    return pl.pallas_call(
        _node_body,
        grid=(_NN // b,),
        in_specs=[pl.BlockSpec((b, _D), lambda i: (i, 0)),
                  pl.BlockSpec((2, b, _D), lambda i: (0, i, 0)),
                  pl.BlockSpec((2 * _D, _D), rep),
                  pl.BlockSpec((1, _D), rep),
                  pl.BlockSpec((_D, _D), rep),
                  pl.BlockSpec((1, _D), rep),
                  pl.BlockSpec((1, _D), rep),
                  pl.BlockSpec((1, _D), rep)],
        out_specs=pl.BlockSpec((b, _D), lambda i: (i, 0)),
        out_shape=jax.ShapeDtypeStruct((_NN, _D), jnp.float32),
    )(node, agg2, w1, b1, w2, b2, ln_g, ln_b)


def _vadd_pack(ba, bb, gb):
    def vrow(r, carry):
        for j in range(_D // 32):
            lo = pl.ds(32 * j, 16)
            hi = pl.ds(32 * j + 16, 16)
            a = ba[r, lo] + bb[r, lo]
            b = ba[r, hi] + bb[r, hi]
            gb[r, pl.ds(32 * j, 32)] = plsc.pack(
                a, b, format=plsc.PackFormat.INTERLEAVED)
        return carry
    lax.fori_loop(0, _CH, vrow, 0)


def _sc_gather_slice(p, q, s_idx, r_idx, ks):
    mesh = plsc.VectorSubcoreMesh(core_axis_name="c", subcore_axis_name="s")

    @functools.partial(
        pl.kernel, mesh=mesh,
        out_type=jax.ShapeDtypeStruct((_SL, _D), jnp.bfloat16),
        scratch_types=[pltpu.VMEM((_CH,), jnp.int32),
                       pltpu.VMEM((_CH,), jnp.int32),
                       pltpu.VMEM((_CH,), jnp.int32),
                       pltpu.VMEM((_CH,), jnp.int32),
                       pltpu.VMEM((_CH, _D), jnp.float32),
                       pltpu.VMEM((_CH, _D), jnp.float32),
                       pltpu.VMEM((_CH, _D), jnp.float32),
                       pltpu.VMEM((_CH, _D), jnp.float32),
                       pltpu.VMEM((_CH, _D), jnp.bfloat16),
                       pltpu.VMEM((_CH, _D), jnp.bfloat16),
                       pltpu.SemaphoreType.DMA,
                       pltpu.SemaphoreType.DMA],
        compiler_params=pltpu.CompilerParams(needs_layout_passes=False),
    )
    def k(p_hbm, q_hbm, s_hbm, r_hbm, g_hbm,
          si0, ri0, si1, ri1, ba0, bb0, ba1, bb1, gb0, gb1, sem0, sem1):
        wid = lax.axis_index("s") * _NC + lax.axis_index("c")
        ibase = ks * _SL + wid * _EPWS
        obase = wid * _EPWS

        def start(chunk, si, ri, ba, bb, sem):
            off = ibase + chunk * _CH
            pltpu.sync_copy(s_hbm.at[pl.ds(off, _CH)], si)
            pltpu.sync_copy(r_hbm.at[pl.ds(off, _CH)], ri)
            pltpu.async_copy(p_hbm.at[si], ba, sem)
            pltpu.async_copy(q_hbm.at[ri], bb, sem)

        def finish(chunk, si, ri, ba, bb, gb, sem):
            pltpu.make_async_copy(p_hbm.at[si], ba, sem).wait()
            pltpu.make_async_copy(q_hbm.at[ri], bb, sem).wait()
            _vadd_pack(ba, bb, gb)
            pltpu.sync_copy(gb, g_hbm.at[pl.ds(obase + chunk * _CH, _CH)])

        start(0, si0, ri0, ba0, bb0, sem0)

        def body(kk, carry):
            c0 = 2 * kk
            start(c0 + 1, si1, ri1, ba1, bb1, sem1)
            finish(c0, si0, ri0, ba0, bb0, gb0, sem0)
            start(c0 + 2, si0, ri0, ba0, bb0, sem0)
            finish(c0 + 1, si1, ri1, ba1, bb1, gb1, sem1)
            return carry

        lax.fori_loop(0, (_NCHS - 1) // 2, body, 0)
        finish(_NCHS - 1, si0, ri0, ba0, bb0, gb0, sem0)

    return k(p, q, s_idx, r_idx)


def _sc_scatter(upds, r_idx, ks0):
    mesh = plsc.VectorSubcoreMesh(core_axis_name="c", subcore_axis_name="s")

    @functools.partial(
        pl.kernel, mesh=mesh,
        out_type=jax.ShapeDtypeStruct((_NC, _NNP, _D), jnp.float32),
        scratch_types=[pltpu.VMEM((_CH,), jnp.int32),
                       pltpu.VMEM((_CH,), jnp.int32),
                       pltpu.VMEM((_CH, _D), jnp.float32),
                       pltpu.VMEM((_CH, _D), jnp.float32),
                       pltpu.SemaphoreType.DMA,
                       pltpu.SemaphoreType.DMA,
                       pltpu.VMEM_SHARED((_NNP, _D), jnp.float32)],
    )
    def k(*refs):
        u_hbms = refs[:len(upds)]
        (r_hbm, o_hbm, ri0, ri1, buf0, buf1,
         sem0, sem1, agg_sh) = refs[len(upds):]
        c = lax.axis_index("c")
        s = lax.axis_index("s")
        wid = s * _NC + c

        def zrow(r, carry):
            for j in range(_D // 16):
                buf0[r, pl.ds(j * 16, 16)] = jnp.zeros((16,), jnp.float32)
            return carry
        lax.fori_loop(0, _CH, zrow, 0)
        for t in range(_RPT // _CH):
            pltpu.sync_copy(buf0, agg_sh.at[pl.ds(s * _RPT + t * _CH, _CH)])
        plsc.subcore_barrier()

        for ku, u_hbm in enumerate(u_hbms):
            ibase = (ks0 + ku) * _SL + wid * _EPWS
            ubase = wid * _EPWS

            def start(chunk, ri, buf, sem):
                pltpu.sync_copy(r_hbm.at[pl.ds(ibase + chunk * _CH, _CH)], ri)
                pltpu.async_copy(u_hbm.at[pl.ds(ubase + chunk * _CH, _CH)],
                                 buf, sem)

            def finish(ri, buf, sem):
                pltpu.make_async_copy(u_hbm.at[pl.ds(ubase, _CH)],
                                      buf, sem).wait()
                pltpu.sync_copy(buf, agg_sh.at[ri], add=True)

            start(0, ri0, buf0, sem0)

            def body(kk, carry):
                c0 = 2 * kk
                start(c0 + 1, ri1, buf1, sem1)
                finish(ri0, buf0, sem0)
                start(c0 + 2, ri0, buf0, sem0)
                finish(ri1, buf1, sem1)
                return carry

            lax.fori_loop(0, (_NCHS - 1) // 2, body, 0)
            finish(ri0, buf0, sem0)

        plsc.subcore_barrier()
        pltpu.sync_copy(agg_sh.at[pl.ds(s * _RPT, _RPT)],
                        o_hbm.at[c, pl.ds(s * _RPT, _RPT)])

    return k(*upds, r_idx)


def _node_body4(n_ref, a_ref, b_ref, w1_ref, b1_ref, w2_ref, b2_ref,
                lg_ref, lb_ref, o_ref):
    n = n_ref[...]
    a = (a_ref[0, :, :] + a_ref[1, :, :]
         + b_ref[0, :, :] + b_ref[1, :, :])
    x = (jnp.dot(n, w1_ref[0:_D, :], preferred_element_type=jnp.float32)
         + jnp.dot(a, w1_ref[_D:2 * _D, :], preferred_element_type=jnp.float32)
         + b1_ref[...])
    h = jnp.maximum(x, 0.0)
    o = jnp.dot(h, w2_ref[...], preferred_element_type=jnp.float32) + b2_ref[...]
    mu = jnp.mean(o, axis=-1, keepdims=True)
    oc = o - mu
    var = jnp.mean(oc * oc, axis=-1, keepdims=True)
    o_ref[...] = oc * lax.rsqrt(var + 1e-5) * lg_ref[...] + lb_ref[...] + n


def _node_mlp4(node, agg_a, agg_b, w1, b1, w2, b2, ln_g, ln_b):
    b = 2000
    rep = lambda i: (0, 0)
    agg_spec = pl.BlockSpec((2, b, _D), lambda i: (0, i, 0))
    return pl.pallas_call(
        _node_body4,
        grid=(_NN // b,),
        in_specs=[pl.BlockSpec((b, _D), lambda i: (i, 0)),
                  agg_spec, agg_spec,
                  pl.BlockSpec((2 * _D, _D), rep),
                  pl.BlockSpec((1, _D), rep),
                  pl.BlockSpec((_D, _D), rep),
                  pl.BlockSpec((1, _D), rep),
                  pl.BlockSpec((1, _D), rep),
                  pl.BlockSpec((1, _D), rep)],
        out_specs=pl.BlockSpec((b, _D), lambda i: (i, 0)),
        out_shape=jax.ShapeDtypeStruct((_NN, _D), jnp.float32),
    )(node, agg_a, agg_b, w1, b1, w2, b2, ln_g, ln_b)


def kernel(node_features, mesh_edge_features, senders, receivers,
           edge_params, node_params):
    senders = senders.astype(jnp.int32)
    receivers = receivers.astype(jnp.int32)
    w1e = edge_params['w1']
    row = lambda v: v.reshape(1, _D)

    p, q = _premul(node_features, w1e[:2 * _D])

    w1c_swz = w1e[2 * _D:][:, _PERM]
    b1_swz = edge_params['b1'][_PERM]
    w2_swz = edge_params['w2'][_PERM, :]

    upds = []
    new_edge = None
    for ks in range(_K):
        g = _sc_gather_slice(p, q, senders, receivers, ks)
        upd_k, new_edge = _edge_mlp_slice(
            g, mesh_edge_features, w1c_swz,
            row(b1_swz), w2_swz, row(edge_params['b2']),
            row(edge_params['ln_g']), row(edge_params['ln_b']),
            ks, new_edge)
        upds.append(upd_k)

    agg_a = _sc_scatter(upds[:3], receivers, 0)
    agg_b = _sc_scatter(upds[3:], receivers, 3)
    new_node = _node_mlp4(
        node_features, agg_a, agg_b, node_params['w1'], row(node_params['b1']),
        node_params['w2'], row(node_params['b2']),
        row(node_params['ln_g']), row(node_params['ln_b']))
    return new_node, new_edge


# speedup vs baseline: 4.2113x; 1.0194x over previous
import functools

import jax
import jax.numpy as jnp
from jax import lax
from jax.experimental import pallas as pl
from jax.experimental.pallas import tpu as pltpu
from jax.experimental.pallas import tpu_sc as plsc

_NN = 10000
_NE = 320000
_D = 128
_CH = 80
_NC = 2
_NS = 16
_NW = _NC * _NS
_U = _NW * _CH
_UNITS = (12, 30, 30, 30, 23)
_K = len(_UNITS)
_SLS = tuple(u * _U for u in _UNITS)
_OFFS = tuple(sum(_SLS[:k]) for k in range(_K))
_EB = 512
_NNP = 10240
_RPT = _NNP // _NS


def _premul_body(n_ref, w_ref, p_ref, q_ref):
    n = n_ref[...]
    p_ref[...] = jnp.dot(n, w_ref[0:_D, :], preferred_element_type=jnp.float32)
    q_ref[...] = jnp.dot(n, w_ref[_D:2 * _D, :], preferred_element_type=jnp.float32)


def _premul(node, w1ab):
    b = 2000
    return pl.pallas_call(
        _premul_body,
        grid=(_NN // b,),
        in_specs=[pl.BlockSpec((b, _D), lambda i: (i, 0)),
                  pl.BlockSpec((2 * _D, _D), lambda i: (0, 0))],
        out_specs=[pl.BlockSpec((b, _D), lambda i: (i, 0)),
                   pl.BlockSpec((b, _D), lambda i: (i, 0))],
        out_shape=[jax.ShapeDtypeStruct((_NN, _D), jnp.float32),
                   jax.ShapeDtypeStruct((_NN, _D), jnp.float32)],
    )(node, w1ab)


def _edge_body(g_ref, e_ref, w1c_ref, b1_ref, w2_ref, b2_ref,
               lg_ref, lb_ref, *rest):
    u_ref, ne_ref = rest[-2], rest[-1]
    e = e_ref[...]
    x = (g_ref[...] + b1_ref[...]
         + jnp.dot(e, w1c_ref[...], preferred_element_type=jnp.float32))
    h = jnp.maximum(x, 0.0)
    o = jnp.dot(h, w2_ref[...], preferred_element_type=jnp.float32) + b2_ref[...]
    mu = jnp.mean(o, axis=-1, keepdims=True)
    oc = o - mu
    var = jnp.mean(oc * oc, axis=-1, keepdims=True)
    u = oc * lax.rsqrt(var + 1e-5) * lg_ref[...] + lb_ref[...]
    u_ref[...] = u
    ne_ref[...] = u + e


def _edge_mlp_slice(g, e_feat, w1c, b1, w2, b2, ln_g, ln_b, ks, ne_alias):
    base = _OFFS[ks] // _EB
    nblk = _SLS[ks] // _EB
    row_l = lambda i: (i, 0)
    row_g = lambda i, base=base: (i + base, 0)
    rep = lambda i: (0, 0)
    ins = [g, e_feat, w1c, b1, w2, b2, ln_g, ln_b]
    in_specs = [pl.BlockSpec((_EB, _D), row_l),
                pl.BlockSpec((_EB, _D), row_g),
                pl.BlockSpec((_D, _D), rep),
                pl.BlockSpec((1, _D), rep),
                pl.BlockSpec((_D, _D), rep),
                pl.BlockSpec((1, _D), rep),
                pl.BlockSpec((1, _D), rep),
                pl.BlockSpec((1, _D), rep)]
    aliases = {}
    if ne_alias is not None:
        ins.append(ne_alias)
        in_specs.append(pl.BlockSpec(memory_space=pl.ANY))
        aliases = {8: 1}
    return pl.pallas_call(
        _edge_body,
        grid=(nblk,),
        in_specs=in_specs,
        out_specs=[pl.BlockSpec((_EB, _D), row_l),
                   pl.BlockSpec((_EB, _D), row_g)],
        out_shape=[jax.ShapeDtypeStruct((_SLS[ks], _D), jnp.float32),
                   jax.ShapeDtypeStruct((_NE, _D), jnp.float32)],
        input_output_aliases=aliases,
    )(*ins)


def _pipe(nch, start, finish):
    start(0, 0)

    def body(kk, carry):
        c0 = 2 * kk
        start(c0 + 1, 1)
        finish(c0, 0)
        start(c0 + 2, 0)
        finish(c0 + 1, 1)
        return carry

    if nch % 2 == 1:
        lax.fori_loop(0, (nch - 1) // 2, body, 0)
        finish(nch - 1, 0)
    else:
        lax.fori_loop(0, (nch - 2) // 2, body, 0)
        start(nch - 1, 1)
        finish(nch - 2, 0)
        finish(nch - 1, 1)


def _vadd_into(ba, bb):
    def vrow(r, carry):
        for j in range(_D // 16):
            sl = pl.ds(j * 16, 16)
            ba[r, sl] = ba[r, sl] + bb[r, sl]
        return carry
    lax.fori_loop(0, _CH, vrow, 0)


def _sc_gather_slice(p, q, s_idx, r_idx, ks):
    mesh = plsc.VectorSubcoreMesh(core_axis_name="c", subcore_axis_name="s")
    nch = _UNITS[ks]
    epw = nch * _CH

    @functools.partial(
        pl.kernel, mesh=mesh,
        out_type=jax.ShapeDtypeStruct((_SLS[ks], _D), jnp.float32),
        scratch_types=[pltpu.VMEM((_CH,), jnp.int32),
                       pltpu.VMEM((_CH,), jnp.int32),
                       pltpu.VMEM((_CH,), jnp.int32),
                       pltpu.VMEM((_CH,), jnp.int32),
                       pltpu.VMEM((_CH, _D), jnp.float32),
                       pltpu.VMEM((_CH, _D), jnp.float32),
                       pltpu.VMEM((_CH, _D), jnp.float32),
                       pltpu.VMEM((_CH, _D), jnp.float32),
                       pltpu.SemaphoreType.DMA,
                       pltpu.SemaphoreType.DMA],
    )
    def k(p_hbm, q_hbm, s_hbm, r_hbm, g_hbm,
          si0, ri0, si1, ri1, ba0, bb0, ba1, bb1, sem0, sem1):
        wid = lax.axis_index("s") * _NC + lax.axis_index("c")
        ibase = _OFFS[ks] + wid * epw
        obase = wid * epw
        slots = ((si0, ri0, ba0, bb0, sem0),
                 (si1, ri1, ba1, bb1, sem1))

        def start(chunk, slot):
            sis, ris, ba, bb, sem = slots[slot]
            off = ibase + chunk * _CH
            pltpu.sync_copy(s_hbm.at[pl.ds(off, _CH)], sis)
            pltpu.sync_copy(r_hbm.at[pl.ds(off, _CH)], ris)
            pltpu.async_copy(p_hbm.at[sis], ba, sem)
            pltpu.async_copy(q_hbm.at[ris], bb, sem)

        def finish(chunk, slot):
            sis, ris, ba, bb, sem = slots[slot]
            pltpu.make_async_copy(p_hbm.at[sis], ba, sem).wait()
            pltpu.make_async_copy(q_hbm.at[ris], bb, sem).wait()
            _vadd_into(ba, bb)
            pltpu.sync_copy(ba, g_hbm.at[pl.ds(obase + chunk * _CH, _CH)])

        _pipe(nch, start, finish)

    return k(p, q, s_idx, r_idx)


def _sc_scatter(upds, r_idx, ks0):
    mesh = plsc.VectorSubcoreMesh(core_axis_name="c", subcore_axis_name="s")

    @functools.partial(
        pl.kernel, mesh=mesh,
        out_type=jax.ShapeDtypeStruct((_NC, _NNP, _D), jnp.float32),
        scratch_types=[pltpu.VMEM((_CH,), jnp.int32),
                       pltpu.VMEM((_CH,), jnp.int32),
                       pltpu.VMEM((_CH, _D), jnp.float32),
                       pltpu.VMEM((_CH, _D), jnp.float32),
                       pltpu.SemaphoreType.DMA,
                       pltpu.SemaphoreType.DMA,
                       pltpu.VMEM_SHARED((_NNP, _D), jnp.float32)],
    )
    def k(*refs):
        u_hbms = refs[:len(upds)]
        (r_hbm, o_hbm, ri0, ri1, buf0, buf1,
         sem0, sem1, agg_sh) = refs[len(upds):]
        c = lax.axis_index("c")
        s = lax.axis_index("s")
        wid = s * _NC + c

        def zrow(r, carry):
            for j in range(_D // 16):
                buf0[r, pl.ds(j * 16, 16)] = jnp.zeros((16,), jnp.float32)
            return carry
        lax.fori_loop(0, _CH, zrow, 0)
        for t in range(_RPT // _CH):
            pltpu.sync_copy(buf0, agg_sh.at[pl.ds(s * _RPT + t * _CH, _CH)])
        plsc.subcore_barrier()

        slots = ((ri0, buf0, sem0), (ri1, buf1, sem1))
        for ku, u_hbm in enumerate(u_hbms):
            ks = ks0 + ku
            epw = _UNITS[ks] * _CH
            ibase = _OFFS[ks] + wid * epw
            ubase = wid * epw

            def start(chunk, slot, u_hbm=u_hbm, ibase=ibase, ubase=ubase):
                ri, buf, sem = slots[slot]
                pltpu.sync_copy(r_hbm.at[pl.ds(ibase + chunk * _CH, _CH)], ri)
                pltpu.async_copy(u_hbm.at[pl.ds(ubase + chunk * _CH, _CH)],
                                 buf, sem)

            def finish(chunk, slot, u_hbm=u_hbm, ubase=ubase):
                ri, buf, sem = slots[slot]
                pltpu.make_async_copy(u_hbm.at[pl.ds(ubase, _CH)],
                                      buf, sem).wait()
                pltpu.sync_copy(buf, agg_sh.at[ri], add=True)

            _pipe(_UNITS[ks], start, finish)

        plsc.subcore_barrier()
        pltpu.sync_copy(agg_sh.at[pl.ds(s * _RPT, _RPT)],
                        o_hbm.at[c, pl.ds(s * _RPT, _RPT)])

    return k(*upds, r_idx)


def _node_body4(n_ref, *rest):
    aggs = rest[:-7]
    w1_ref, b1_ref, w2_ref, b2_ref, lg_ref, lb_ref, o_ref = rest[-7:]
    n = n_ref[...]
    a = aggs[0][0, :, :] + aggs[0][1, :, :]
    for ar in aggs[1:]:
        a = a + ar[0, :, :] + ar[1, :, :]
    x = (jnp.dot(n, w1_ref[0:_D, :], preferred_element_type=jnp.float32)
         + jnp.dot(a, w1_ref[_D:2 * _D, :], preferred_element_type=jnp.float32)
         + b1_ref[...])
    h = jnp.maximum(x, 0.0)
    o = jnp.dot(h, w2_ref[...], preferred_element_type=jnp.float32) + b2_ref[...]
    mu = jnp.mean(o, axis=-1, keepdims=True)
    oc = o - mu
    var = jnp.mean(oc * oc, axis=-1, keepdims=True)
    o_ref[...] = oc * lax.rsqrt(var + 1e-5) * lg_ref[...] + lb_ref[...] + n


def _node_mlp4(node, aggs, w1, b1, w2, b2, ln_g, ln_b):
    b = 2000
    rep = lambda i: (0, 0)
    agg_spec = pl.BlockSpec((2, b, _D), lambda i: (0, i, 0))
    return pl.pallas_call(
        _node_body4,
        grid=(_NN // b,),
        in_specs=[pl.BlockSpec((b, _D), lambda i: (i, 0))]
                 + [agg_spec] * len(aggs)
                 + [pl.BlockSpec((2 * _D, _D), rep),
                    pl.BlockSpec((1, _D), rep),
                    pl.BlockSpec((_D, _D), rep),
                    pl.BlockSpec((1, _D), rep),
                    pl.BlockSpec((1, _D), rep),
                    pl.BlockSpec((1, _D), rep)],
        out_specs=pl.BlockSpec((b, _D), lambda i: (i, 0)),
        out_shape=jax.ShapeDtypeStruct((_NN, _D), jnp.float32),
    )(node, *aggs, w1, b1, w2, b2, ln_g, ln_b)


def kernel(node_features, mesh_edge_features, senders, receivers,
           edge_params, node_params):
    senders = senders.astype(jnp.int32)
    receivers = receivers.astype(jnp.int32)
    w1e = edge_params['w1']
    row = lambda v: v.reshape(1, _D)

    p, q = _premul(node_features, w1e[:2 * _D])

    upds = []
    new_edge = None
    for ks in range(_K):
        g = _sc_gather_slice(p, q, senders, receivers, ks)
        upd_k, new_edge = _edge_mlp_slice(
            g, mesh_edge_features, w1e[2 * _D:],
            row(edge_params['b1']), edge_params['w2'], row(edge_params['b2']),
            row(edge_params['ln_g']), row(edge_params['ln_b']),
            ks, new_edge)
        upds.append(upd_k)

    aggs = [_sc_scatter(upds[:3], receivers, 0),
            _sc_scatter(upds[3:4], receivers, 3),
            _sc_scatter(upds[4:], receivers, 4)]
    new_node = _node_mlp4(
        node_features, aggs, node_params['w1'], row(node_params['b1']),
        node_params['w2'], row(node_params['b2']),
        row(node_params['ln_g']), row(node_params['ln_b']))
    return new_node, new_edge


# speedup vs baseline: 5.3180x; 1.2628x over previous
import functools

import jax
import jax.numpy as jnp
from jax import lax
from jax.experimental import pallas as pl
from jax.experimental.pallas import tpu as pltpu
from jax.experimental.pallas import tpu_sc as plsc

_NN = 10000
_NE = 320000
_D = 128
_CH = 80
_NC = 2
_NS = 16
_NW = _NC * _NS
_U = _NW * _CH
_UNITS = (12, 30, 30, 30, 23)
_K = len(_UNITS)
_SLS = tuple(u * _U for u in _UNITS)
_OFFS = tuple(sum(_SLS[:k]) for k in range(_K))
_EB = 1280
_NNP = 10240
_RPT = _NNP // _NS


def _premul_body(n_ref, w_ref, p_ref, q_ref):
    n = n_ref[...]
    p_ref[...] = jnp.dot(n, w_ref[0:_D, :], preferred_element_type=jnp.float32)
    q_ref[...] = jnp.dot(n, w_ref[_D:2 * _D, :], preferred_element_type=jnp.float32)


def _premul(node, w1ab):
    b = 2000
    return pl.pallas_call(
        _premul_body,
        grid=(_NN // b,),
        in_specs=[pl.BlockSpec((b, _D), lambda i: (i, 0)),
                  pl.BlockSpec((2 * _D, _D), lambda i: (0, 0))],
        out_specs=[pl.BlockSpec((b, _D), lambda i: (i, 0)),
                   pl.BlockSpec((b, _D), lambda i: (i, 0))],
        out_shape=[jax.ShapeDtypeStruct((_NN, _D), jnp.float32),
                   jax.ShapeDtypeStruct((_NN, _D), jnp.float32)],
    )(node, w1ab)


def _edge_body(g_ref, e_ref, w1c_ref, b1_ref, w2_ref, b2_ref,
               lg_ref, lb_ref, *rest):
    u_ref, ne_ref = rest[-2], rest[-1]
    e = e_ref[...]
    x = (g_ref[...] + b1_ref[...]
         + jnp.dot(e, w1c_ref[...], preferred_element_type=jnp.float32))
    h = jnp.maximum(x, 0.0)
    o = jnp.dot(h, w2_ref[...], preferred_element_type=jnp.float32) + b2_ref[...]
    mu = jnp.mean(o, axis=-1, keepdims=True)
    oc = o - mu
    var = jnp.mean(oc * oc, axis=-1, keepdims=True)
    u = oc * lax.rsqrt(var + 1e-5) * lg_ref[...] + lb_ref[...]
    u_ref[...] = u
    ne_ref[...] = u + e


def _edge_mlp_slice(g, e_feat, w1c, b1, w2, b2, ln_g, ln_b, ks, ne_alias):
    base = _OFFS[ks] // _EB
    nblk = _SLS[ks] // _EB
    row_l = lambda i: (i, 0)
    row_g = lambda i, base=base: (i + base, 0)
    rep = lambda i: (0, 0)
    ins = [g, e_feat, w1c, b1, w2, b2, ln_g, ln_b]
    in_specs = [pl.BlockSpec((_EB, _D), row_l),
                pl.BlockSpec((_EB, _D), row_g),
                pl.BlockSpec((_D, _D), rep),
                pl.BlockSpec((1, _D), rep),
                pl.BlockSpec((_D, _D), rep),
                pl.BlockSpec((1, _D), rep),
                pl.BlockSpec((1, _D), rep),
                pl.BlockSpec((1, _D), rep)]
    aliases = {}
    if ne_alias is not None:
        ins.append(ne_alias)
        in_specs.append(pl.BlockSpec(memory_space=pl.ANY))
        aliases = {8: 1}
    return pl.pallas_call(
        _edge_body,
        grid=(nblk,),
        in_specs=in_specs,
        out_specs=[pl.BlockSpec((_EB, _D), row_l),
                   pl.BlockSpec((_EB, _D), row_g)],
        out_shape=[jax.ShapeDtypeStruct((_SLS[ks], _D), jnp.float32),
                   jax.ShapeDtypeStruct((_NE, _D), jnp.float32)],
        input_output_aliases=aliases,
    )(*ins)


def _pipe(nch, start, finish):
    start(0, 0)

    def body(kk, carry):
        c0 = 2 * kk
        start(c0 + 1, 1)
        finish(c0, 0)
        start(c0 + 2, 0)
        finish(c0 + 1, 1)
        return carry

    if nch % 2 == 1:
        lax.fori_loop(0, (nch - 1) // 2, body, 0)
        finish(nch - 1, 0)
    else:
        lax.fori_loop(0, (nch - 2) // 2, body, 0)
        start(nch - 1, 1)
        finish(nch - 2, 0)
        finish(nch - 1, 1)


def _vadd_into(ba, bb):
    def vrow(r, carry):
        for j in range(_D // 16):
            sl = pl.ds(j * 16, 16)
            ba[r, sl] = ba[r, sl] + bb[r, sl]
        return carry
    lax.fori_loop(0, _CH, vrow, 0)


def _sc_gather_slice(p, q, s_idx, r_idx, ks):
    mesh = plsc.VectorSubcoreMesh(core_axis_name="c", subcore_axis_name="s")
    nch = _UNITS[ks]
    epw = nch * _CH

    @functools.partial(
        pl.kernel, mesh=mesh,
        out_type=jax.ShapeDtypeStruct((_SLS[ks], _D), jnp.float32),
        scratch_types=[pltpu.VMEM((_CH,), jnp.int32),
                       pltpu.VMEM((_CH,), jnp.int32),
                       pltpu.VMEM((_CH,), jnp.int32),
                       pltpu.VMEM((_CH,), jnp.int32),
                       pltpu.VMEM((_CH, _D), jnp.float32),
                       pltpu.VMEM((_CH, _D), jnp.float32),
                       pltpu.VMEM((_CH, _D), jnp.float32),
                       pltpu.VMEM((_CH, _D), jnp.float32),
                       pltpu.SemaphoreType.DMA,
                       pltpu.SemaphoreType.DMA],
    )
    def k(p_hbm, q_hbm, s_hbm, r_hbm, g_hbm,
          si0, ri0, si1, ri1, ba0, bb0, ba1, bb1, sem0, sem1):
        wid = lax.axis_index("s") * _NC + lax.axis_index("c")
        ibase = _OFFS[ks] + wid * epw
        obase = wid * epw
        slots = ((si0, ri0, ba0, bb0, sem0),
                 (si1, ri1, ba1, bb1, sem1))

        def start(chunk, slot):
            sis, ris, ba, bb, sem = slots[slot]
            off = ibase + chunk * _CH
            pltpu.sync_copy(s_hbm.at[pl.ds(off, _CH)], sis)
            pltpu.sync_copy(r_hbm.at[pl.ds(off, _CH)], ris)
            pltpu.async_copy(p_hbm.at[sis], ba, sem)
            pltpu.async_copy(q_hbm.at[ris], bb, sem)

        def finish(chunk, slot):
            sis, ris, ba, bb, sem = slots[slot]
            pltpu.make_async_copy(p_hbm.at[sis], ba, sem).wait()
            pltpu.make_async_copy(q_hbm.at[ris], bb, sem).wait()
            _vadd_into(ba, bb)
            pltpu.sync_copy(ba, g_hbm.at[pl.ds(obase + chunk * _CH, _CH)])

        _pipe(nch, start, finish)

    return k(p, q, s_idx, r_idx)


def _sc_scatter(upds, r_idx, ks0):
    mesh = plsc.VectorSubcoreMesh(core_axis_name="c", subcore_axis_name="s")

    @functools.partial(
        pl.kernel, mesh=mesh,
        out_type=jax.ShapeDtypeStruct((_NC, _NNP, _D), jnp.float32),
        scratch_types=[pltpu.VMEM((_CH,), jnp.int32),
                       pltpu.VMEM((_CH,), jnp.int32),
                       pltpu.VMEM((_CH, _D), jnp.float32),
                       pltpu.VMEM((_CH, _D), jnp.float32),
                       pltpu.SemaphoreType.DMA,
                       pltpu.SemaphoreType.DMA,
                       pltpu.VMEM_SHARED((_NNP, _D), jnp.float32)],
    )
    def k(*refs):
        u_hbms = refs[:len(upds)]
        (r_hbm, o_hbm, ri0, ri1, buf0, buf1,
         sem0, sem1, agg_sh) = refs[len(upds):]
        c = lax.axis_index("c")
        s = lax.axis_index("s")
        wid = s * _NC + c

        def zrow(r, carry):
            for j in range(_D // 16):
                buf0[r, pl.ds(j * 16, 16)] = jnp.zeros((16,), jnp.float32)
            return carry
        lax.fori_loop(0, _CH, zrow, 0)
        for t in range(_RPT // _CH):
            pltpu.sync_copy(buf0, agg_sh.at[pl.ds(s * _RPT + t * _CH, _CH)])
        plsc.subcore_barrier()

        slots = ((ri0, buf0, sem0), (ri1, buf1, sem1))
        for ku, u_hbm in enumerate(u_hbms):
            ks = ks0 + ku
            epw = _UNITS[ks] * _CH
            ibase = _OFFS[ks] + wid * epw
            ubase = wid * epw

            def start(chunk, slot, u_hbm=u_hbm, ibase=ibase, ubase=ubase):
                ri, buf, sem = slots[slot]
                pltpu.sync_copy(r_hbm.at[pl.ds(ibase + chunk * _CH, _CH)], ri)
                pltpu.async_copy(u_hbm.at[pl.ds(ubase + chunk * _CH, _CH)],
                                 buf, sem)

            def finish(chunk, slot, u_hbm=u_hbm, ubase=ubase):
                ri, buf, sem = slots[slot]
                pltpu.make_async_copy(u_hbm.at[pl.ds(ubase, _CH)],
                                      buf, sem).wait()
                pltpu.sync_copy(buf, agg_sh.at[ri], add=True)

            _pipe(_UNITS[ks], start, finish)

        plsc.subcore_barrier()
        pltpu.sync_copy(agg_sh.at[pl.ds(s * _RPT, _RPT)],
                        o_hbm.at[c, pl.ds(s * _RPT, _RPT)])

    return k(*upds, r_idx)


def _node_body4(n_ref, *rest):
    aggs = rest[:-7]
    w1_ref, b1_ref, w2_ref, b2_ref, lg_ref, lb_ref, o_ref = rest[-7:]
    n = n_ref[...]
    a = aggs[0][0, :, :] + aggs[0][1, :, :]
    for ar in aggs[1:]:
        a = a + ar[0, :, :] + ar[1, :, :]
    x = (jnp.dot(n, w1_ref[0:_D, :], preferred_element_type=jnp.float32)
         + jnp.dot(a, w1_ref[_D:2 * _D, :], preferred_element_type=jnp.float32)
         + b1_ref[...])
    h = jnp.maximum(x, 0.0)
    o = jnp.dot(h, w2_ref[...], preferred_element_type=jnp.float32) + b2_ref[...]
    mu = jnp.mean(o, axis=-1, keepdims=True)
    oc = o - mu
    var = jnp.mean(oc * oc, axis=-1, keepdims=True)
    o_ref[...] = oc * lax.rsqrt(var + 1e-5) * lg_ref[...] + lb_ref[...] + n


def _node_mlp4(node, aggs, w1, b1, w2, b2, ln_g, ln_b):
    b = 2000
    rep = lambda i: (0, 0)
    agg_spec = pl.BlockSpec((2, b, _D), lambda i: (0, i, 0))
    return pl.pallas_call(
        _node_body4,
        grid=(_NN // b,),
        in_specs=[pl.BlockSpec((b, _D), lambda i: (i, 0))]
                 + [agg_spec] * len(aggs)
                 + [pl.BlockSpec((2 * _D, _D), rep),
                    pl.BlockSpec((1, _D), rep),
                    pl.BlockSpec((_D, _D), rep),
                    pl.BlockSpec((1, _D), rep),
                    pl.BlockSpec((1, _D), rep),
                    pl.BlockSpec((1, _D), rep)],
        out_specs=pl.BlockSpec((b, _D), lambda i: (i, 0)),
        out_shape=jax.ShapeDtypeStruct((_NN, _D), jnp.float32),
    )(node, *aggs, w1, b1, w2, b2, ln_g, ln_b)


def kernel(node_features, mesh_edge_features, senders, receivers,
           edge_params, node_params):
    senders = senders.astype(jnp.int32)
    receivers = receivers.astype(jnp.int32)
    w1e = edge_params['w1']
    row = lambda v: v.reshape(1, _D)

    p, q = _premul(node_features, w1e[:2 * _D])

    upds = []
    new_edge = None
    for ks in range(_K):
        g = _sc_gather_slice(p, q, senders, receivers, ks)
        upd_k, new_edge = _edge_mlp_slice(
            g, mesh_edge_features, w1e[2 * _D:],
            row(edge_params['b1']), edge_params['w2'], row(edge_params['b2']),
            row(edge_params['ln_g']), row(edge_params['ln_b']),
            ks, new_edge)
        upds.append(upd_k)

    aggs = [_sc_scatter(upds[:3], receivers, 0),
            _sc_scatter(upds[3:4], receivers, 3),
            _sc_scatter(upds[4:], receivers, 4)]
    new_node = _node_mlp4(
        node_features, aggs, node_params['w1'], row(node_params['b1']),
        node_params['w2'], row(node_params['b2']),
        row(node_params['ln_g']), row(node_params['ln_b']))
    return new_node, new_edge
